# baseline (speedup 1.0000x reference)
import functools

import jax
import jax.numpy as jnp
from jax import lax
from jax.experimental import pallas as pl
from jax.experimental.pallas import tpu as pltpu
from jax.experimental.pallas import tpu_sc as plsc

LAMBDA_CC = 10.0
BINS = 64
EPS = 1e-06

NC, NS, L = 2, 16, 16
NW = NC * NS
B, C, H, W = 16, 3, 512, 512
HW = H * W
HALF = HW // NC
PR = 16
P = PR * W
NCH = HALF // P
NSETS = 4
VECS = P // L
CW = W // L
HL = BINS * L
SL = C * L


def _sc_body(x_hbm, y_hbm, xh_out, yh_out, xs_out, ys_out,
             b0r, b0g, b0b, b1r, b1g, b1b, b2r, b2g, b2b, b3r, b3g, b3b,
             histl_v, sums_v, sem_a, sem_b, sem_c, sem_d):
    bufs = ((b0r, b0g, b0b), (b1r, b1g, b1b), (b2r, b2g, b2b), (b3r, b3g, b3b))
    sems = (sem_a, sem_b, sem_c, sem_d)
    cid = lax.axis_index("c")
    sid = lax.axis_index("s")
    wid = cid * NS + sid
    base = cid * HALF
    zero16 = jnp.zeros((L,), jnp.float32)
    ones16 = jnp.ones((L,), jnp.float32)
    lane = lax.iota(jnp.int32, L)

    row0 = cid * (H // NC)

    def start_copies(src, chunk):
        bset = chunk % NSETS
        return [
            pltpu.async_copy(
                src.at[sid * C + ch, pl.ds(row0 + chunk * PR, PR), :],
                bufs[bset][ch], sems[bset])
            for ch in range(C)
        ]

    def accum_chunk(bset, sums):
        def vbody(v, carry):
            sr, sg, sb = carry
            rw = lax.shift_right_logical(v, 5)
            col = lax.shift_left(lax.bitwise_and(v, CW - 1), 4)
            r = (bufs[bset][0][rw, pl.ds(col, L)] + 1.0) * 0.5
            g = (bufs[bset][1][rw, pl.ds(col, L)] + 1.0) * 0.5
            b = (bufs[bset][2][rw, pl.ds(col, L)] + 1.0) * 0.5
            lum = r * 0.299 + g * 0.587 + b * 0.114
            t = lum * 64.0
            ti = t.astype(jnp.int32)
            tf = ti.astype(jnp.float32)
            idx = ti - jnp.where(tf == t, 1, 0)
            idx = jnp.minimum(jnp.maximum(idx, 0), BINS - 1)
            plsc.addupdate_scatter(histl_v, [idx * L + lane], ones16)
            return (sr + r, sg + g, sb + b)

        return plsc.parallel_loop(0, VECS, 1, unroll=4, carry=sums)(vbody)

    def process(src, h_out, s_out):
        for i in range(BINS):
            histl_v[pl.ds(i * L, L)] = zero16
        pend = {j: start_copies(src, j) for j in range(min(NSETS, NCH))}
        sums = (zero16,) * 3
        for chunk in range(NCH):
            for h in pend.pop(chunk):
                h.wait()
            sums = accum_chunk(chunk % NSETS, sums)
            if chunk + NSETS < NCH:
                pend[chunk + NSETS] = start_copies(src, chunk + NSETS)
        sums_v[pl.ds(0, L)] = sums[0]
        sums_v[pl.ds(L, L)] = sums[1]
        sums_v[pl.ds(2 * L, L)] = sums[2]
        pltpu.sync_copy(histl_v, h_out.at[pl.ds(wid * HL, HL)])
        pltpu.sync_copy(sums_v, s_out.at[pl.ds(wid * SL, SL)])

    process(x_hbm, xh_out, xs_out)
    process(y_hbm, yh_out, ys_out)


_sc_hist = functools.partial(
    pl.kernel,
    out_type=[
        jax.ShapeDtypeStruct((NW * HL,), jnp.float32),
        jax.ShapeDtypeStruct((NW * HL,), jnp.float32),
        jax.ShapeDtypeStruct((NW * SL,), jnp.float32),
        jax.ShapeDtypeStruct((NW * SL,), jnp.float32),
    ],
    mesh=plsc.VectorSubcoreMesh(
        core_axis_name="c", subcore_axis_name="s", num_cores=NC, num_subcores=NS
    ),
    compiler_params=pltpu.CompilerParams(needs_layout_passes=False),
    scratch_types=[
        pltpu.VMEM((PR, W), jnp.float32),
        pltpu.VMEM((PR, W), jnp.float32),
        pltpu.VMEM((PR, W), jnp.float32),
        pltpu.VMEM((PR, W), jnp.float32),
        pltpu.VMEM((PR, W), jnp.float32),
        pltpu.VMEM((PR, W), jnp.float32),
        pltpu.VMEM((PR, W), jnp.float32),
        pltpu.VMEM((PR, W), jnp.float32),
        pltpu.VMEM((PR, W), jnp.float32),
        pltpu.VMEM((PR, W), jnp.float32),
        pltpu.VMEM((PR, W), jnp.float32),
        pltpu.VMEM((PR, W), jnp.float32),
        pltpu.VMEM((HL,), jnp.float32),
        pltpu.VMEM((SL,), jnp.float32),
        pltpu.SemaphoreType.DMA,
        pltpu.SemaphoreType.DMA,
        pltpu.SemaphoreType.DMA,
        pltpu.SemaphoreType.DMA,
    ],
)(_sc_body)


def _tail_body(xh_ref, yh_ref, xs_ref, ys_ref, out_ref):
    hw = jnp.float32(HW)
    xh = jnp.sum(xh_ref[0], axis=2) + jnp.sum(xh_ref[1], axis=2)
    yh = jnp.sum(yh_ref[0], axis=2) + jnp.sum(yh_ref[1], axis=2)
    xs = (jnp.sum(xs_ref[0], axis=2) + jnp.sum(xs_ref[1], axis=2)) / hw
    ys = (jnp.sum(ys_ref[0], axis=2) + jnp.sum(ys_ref[1], axis=2)) / hw

    xr, xg, xb = xs[:, 0:1], xs[:, 1:2], xs[:, 2:3]
    yr, yg, yb = ys[:, 0:1], ys[:, 1:2], ys[:, 2:3]
    grey_world = jnp.mean(jnp.abs(xr - xg) + jnp.abs(xg - xb) + jnp.abs(xb - xr))

    xden = xr + xg + xb + EPS
    yden = yr + yg + yb + EPS
    ratio = (jnp.mean(jnp.abs(xr / xden - yr / yden))
             + jnp.mean(jnp.abs(xg / xden - yg / yden))
             + jnp.mean(jnp.abs(xb / xden - yb / yden))) / 3.0

    xn = (xh + EPS) / (jnp.sum(xh, axis=1, keepdims=True) + EPS * BINS)
    yn = (yh + EPS) / (jnp.sum(yh, axis=1, keepdims=True) + EPS * BINS)
    kl = jnp.sum(yn * (jnp.log(yn) - jnp.log(xn))) / B

    out_ref[0, 0] = LAMBDA_CC * (grey_world + ratio + kl)


def kernel(x, y):
    x1 = x.reshape(B * C, H, W)
    y1 = y.reshape(B * C, H, W)
    xh, yh, xs, ys = _sc_hist(x1, y1)
    xh4 = xh.reshape(NC, NS, BINS, L)
    yh4 = yh.reshape(NC, NS, BINS, L)
    xs4 = xs.reshape(NC, NS, C, L)
    ys4 = ys.reshape(NC, NS, C, L)
    loss = pl.pallas_call(
        _tail_body,
        out_shape=jax.ShapeDtypeStruct((1, 1), jnp.float32),
        out_specs=pl.BlockSpec(memory_space=pltpu.SMEM),
    )(xh4, yh4, xs4, ys4)
    return loss[0, 0]


# speedup vs baseline: 1.4109x; 1.4109x over previous
import functools

import jax
import jax.numpy as jnp
from jax import lax
from jax.experimental import pallas as pl
from jax.experimental.pallas import tpu as pltpu
from jax.experimental.pallas import tpu_sc as plsc

LAMBDA_CC = 10.0
BINS = 64
EPS = 1e-06

NC, NS, L = 2, 16, 16
NW = NC * NS
B, C, H, W = 16, 3, 512, 512
HW = H * W
HALF = HW // NC
PR = 32
P = PR * W
NCH = HALF // P
NSETS = 2
VECS = P // L
CW = W // L
HL = BINS * L
SL = C * L


def _sc_body(x_hbm, y_hbm, xh_out, yh_out, xs_out, ys_out,
             b0r, b0g, b0b, b1r, b1g, b1b, histl_v, sums_v,
             sem_a, sem_b):
    bufs = ((b0r, b0g, b0b), (b1r, b1g, b1b))
    sems = (sem_a, sem_b)
    cid = lax.axis_index("c")
    sid = lax.axis_index("s")
    wid = cid * NS + sid
    base = cid * HALF
    zero16 = jnp.zeros((L,), jnp.float32)
    ones16 = jnp.ones((L,), jnp.float32)
    lane = lax.iota(jnp.int32, L)

    row0 = cid * (H // NC)

    def start_copies(src, chunk):
        bset = chunk % NSETS
        return [
            pltpu.async_copy(
                src.at[sid * C + ch, pl.ds(row0 + chunk * PR, PR), :],
                bufs[bset][ch], sems[bset])
            for ch in range(C)
        ]

    W0, W1, W2 = 32.0 * 0.299, 32.0 * 0.587, 32.0 * 0.114

    def accum_chunk(bset, sums):
        def vbody(v, carry):
            sr, sg, sb = carry
            rw = lax.shift_right_logical(v, 5)
            col = lax.shift_left(lax.bitwise_and(v, CW - 1), 4)
            r = bufs[bset][0][rw, pl.ds(col, L)]
            g = bufs[bset][1][rw, pl.ds(col, L)]
            b = bufs[bset][2][rw, pl.ds(col, L)]
            t = (W0 * r + W1 * g) + (W2 * b + 32.0)
            idx = jnp.minimum(jnp.maximum(t.astype(jnp.int32), 0), BINS - 1)
            plsc.addupdate_scatter(histl_v, [idx * L + lane], ones16)
            return (sr + r, sg + g, sb + b)

        return plsc.parallel_loop(0, VECS, 1, unroll=8, carry=sums)(vbody)

    def process(src, h_out, s_out):
        for i in range(BINS):
            histl_v[pl.ds(i * L, L)] = zero16
        pend = {j: start_copies(src, j) for j in range(min(NSETS, NCH))}
        sums = (zero16,) * 3
        for chunk in range(NCH):
            for h in pend.pop(chunk):
                h.wait()
            sums = accum_chunk(chunk % NSETS, sums)
            if chunk + NSETS < NCH:
                pend[chunk + NSETS] = start_copies(src, chunk + NSETS)
        sums_v[pl.ds(0, L)] = sums[0]
        sums_v[pl.ds(L, L)] = sums[1]
        sums_v[pl.ds(2 * L, L)] = sums[2]
        pltpu.sync_copy(histl_v, h_out.at[pl.ds(wid * HL, HL)])
        pltpu.sync_copy(sums_v, s_out.at[pl.ds(wid * SL, SL)])

    process(x_hbm, xh_out, xs_out)
    process(y_hbm, yh_out, ys_out)


_sc_hist = functools.partial(
    pl.kernel,
    out_type=[
        jax.ShapeDtypeStruct((NW * HL,), jnp.float32),
        jax.ShapeDtypeStruct((NW * HL,), jnp.float32),
        jax.ShapeDtypeStruct((NW * SL,), jnp.float32),
        jax.ShapeDtypeStruct((NW * SL,), jnp.float32),
    ],
    mesh=plsc.VectorSubcoreMesh(
        core_axis_name="c", subcore_axis_name="s", num_cores=NC, num_subcores=NS
    ),
    compiler_params=pltpu.CompilerParams(needs_layout_passes=False),
    scratch_types=[
        pltpu.VMEM((PR, W), jnp.float32),
        pltpu.VMEM((PR, W), jnp.float32),
        pltpu.VMEM((PR, W), jnp.float32),
        pltpu.VMEM((PR, W), jnp.float32),
        pltpu.VMEM((PR, W), jnp.float32),
        pltpu.VMEM((PR, W), jnp.float32),
        pltpu.VMEM((HL,), jnp.float32),
        pltpu.VMEM((SL,), jnp.float32),
        pltpu.SemaphoreType.DMA,
        pltpu.SemaphoreType.DMA,
    ],
)(_sc_body)


def _tail_body(xh_ref, yh_ref, xs_ref, ys_ref, out_ref):
    hw = jnp.float32(HW)
    xh = jnp.sum(xh_ref[0], axis=2) + jnp.sum(xh_ref[1], axis=2)
    yh = jnp.sum(yh_ref[0], axis=2) + jnp.sum(yh_ref[1], axis=2)
    xs = (jnp.sum(xs_ref[0], axis=2) + jnp.sum(xs_ref[1], axis=2)) / hw * 0.5 + 0.5
    ys = (jnp.sum(ys_ref[0], axis=2) + jnp.sum(ys_ref[1], axis=2)) / hw * 0.5 + 0.5

    xr, xg, xb = xs[:, 0:1], xs[:, 1:2], xs[:, 2:3]
    yr, yg, yb = ys[:, 0:1], ys[:, 1:2], ys[:, 2:3]
    grey_world = jnp.mean(jnp.abs(xr - xg) + jnp.abs(xg - xb) + jnp.abs(xb - xr))

    xden = xr + xg + xb + EPS
    yden = yr + yg + yb + EPS
    ratio = (jnp.mean(jnp.abs(xr / xden - yr / yden))
             + jnp.mean(jnp.abs(xg / xden - yg / yden))
             + jnp.mean(jnp.abs(xb / xden - yb / yden))) / 3.0

    xn = (xh + EPS) / (jnp.sum(xh, axis=1, keepdims=True) + EPS * BINS)
    yn = (yh + EPS) / (jnp.sum(yh, axis=1, keepdims=True) + EPS * BINS)
    kl = jnp.sum(yn * (jnp.log(yn) - jnp.log(xn))) / B

    out_ref[0, 0] = LAMBDA_CC * (grey_world + ratio + kl)


def kernel(x, y):
    x1 = x.reshape(B * C, H, W)
    y1 = y.reshape(B * C, H, W)
    xh, yh, xs, ys = _sc_hist(x1, y1)
    xh4 = xh.reshape(NC, NS, BINS, L)
    yh4 = yh.reshape(NC, NS, BINS, L)
    xs4 = xs.reshape(NC, NS, C, L)
    ys4 = ys.reshape(NC, NS, C, L)
    loss = pl.pallas_call(
        _tail_body,
        out_shape=jax.ShapeDtypeStruct((1, 1), jnp.float32),
        out_specs=pl.BlockSpec(memory_space=pltpu.SMEM),
    )(xh4, yh4, xs4, ys4)
    return loss[0, 0]
